# baseline (speedup 1.0000x reference)
import functools

import jax
import jax.numpy as jnp
from jax import lax
from jax.experimental import pallas as pl
from jax.experimental.pallas import tpu as pltpu
from jax.experimental.pallas import tpu_sc as plsc

H = 512
HALF = 384

_info = plsc.get_sparse_core_info()
_NC, _NS, _L = _info.num_cores, _info.num_subcores, _info.num_lanes
_NW = _NC * _NS


def _make_gather(n_rows: int):
  R = 40
  NBG = 4
  NBW = 2
  G = n_rows // (_NW * R)
  MAIN = ((G - 2 * NBG) // NBG) * NBG
  assert n_rows == G * _NW * R and G >= 2 * NBG and MAIN > 0
  mesh = plsc.VectorSubcoreMesh(core_axis_name="c", subcore_axis_name="s")

  @functools.partial(
      pl.kernel,
      mesh=mesh,
      out_type=jax.ShapeDtypeStruct((G, _NC, _NS, R, HALF), jnp.float32),
      scratch_types=[
          pltpu.VMEM((G, R), jnp.int32),
          pltpu.VMEM((NBG, R, HALF), jnp.float32),
          pltpu.VMEM_SHARED((NBW, _NS, R, HALF), jnp.float32),
          pltpu.SemaphoreType.DMA,
          pltpu.SemaphoreType.DMA,
          pltpu.SemaphoreType.DMA,
          pltpu.SemaphoreType.DMA,
          pltpu.SemaphoreType.DMA,
          pltpu.SemaphoreType.DMA,
      ],
  )
  def k(table_hbm, idx_hbm, out_hbm, idx_v, rows_v, shared,
        g0, g1, g2, g3, w0, w1):
    cid = lax.axis_index("c")
    sid = lax.axis_index("s")
    gsem = (g0, g1, g2, g3)
    wsem = (w0, w1)

    pltpu.sync_copy(idx_hbm.at[:, cid, sid], idx_v)
    offs = (lax.iota(jnp.int32, _L) & 1) * H

    @pl.loop(0, G)
    def _(g):
      @pl.loop(0, R, step=_L)
      def _(i):
        sl = pl.ds(i, _L)
        idx_v[g, sl] = idx_v[g, sl] + offs

    def start_gather(g, bg):
      return pltpu.async_copy(
          table_hbm.at[idx_v.at[g]], rows_v.at[bg], gsem[bg])

    def wait_gather(bg):
      pltpu.make_async_copy(
          table_hbm.at[idx_v.at[0]], rows_v.at[bg], gsem[bg]).wait()

    def start_write(g, bw):
      return pltpu.async_copy(shared.at[bw], out_hbm.at[g, cid], wsem[bw])

    def wait_write(bw):
      pltpu.make_async_copy(
          shared.at[bw], out_hbm.at[0, cid], wsem[bw]).wait()

    def round_body(g, bg, bw, drain, prefetch):
      wait_gather(bg)
      if drain:
        @pl.when(sid == 0)
        def _():
          wait_write(bw)
      plsc.subcore_barrier()
      pltpu.sync_copy(rows_v.at[bg], shared.at[bw, sid])
      if prefetch:
        start_gather(g + NBG, bg)
      plsc.subcore_barrier()

      @pl.when(sid == 0)
      def _():
        start_write(g, bw)

    for b in range(NBG):
      start_gather(b, b)
    for g in range(NBG):
      round_body(g, g % NBG, g % NBW, drain=(g >= NBW), prefetch=True)

    @pl.loop(NBG, NBG + MAIN, step=NBG)
    def _(o):
      for b in range(NBG):
        round_body(o + b, b, b % NBW, drain=True, prefetch=True)

    for g in range(NBG + MAIN, G):
      round_body(g, g % NBG, g % NBW, drain=True, prefetch=(g + NBG < G))

    @pl.when(sid == 0)
    def _():
      for b in range(NBW):
        wait_write(b)

    plsc.subcore_barrier()

  return k


_gather = _make_gather(2 * 1024 * 50)
_G = 2 * 1024 * 50 // (_NW * 40)


def kernel(token, row_embed, col_embed):
  B, L, _ = token.shape
  table = jnp.concatenate([row_embed, col_embed], axis=0)
  idx = token.astype(jnp.int32).reshape(_G, _NC, _NS, 40)
  out = _gather(table, idx)
  return out.reshape(B, L, 2 * HALF)
